# speedup vs baseline: 1.5843x; 1.5843x over previous
import functools

import jax
import jax.numpy as jnp
from jax import lax
from jax.experimental import pallas as pl
from jax.experimental.pallas import tpu as pltpu
from jax.experimental.pallas import tpu_sc as plsc

_B, _L, _D = 4, 4096, 1024
_N = _B * _L

_NC, _NS = 2, 16
_NW = _NC * _NS
_ROWS_PER_W = _N // _NW
_CHUNK = 64
_NCHUNKS = _ROWS_PER_W // _CHUNK

_mesh = plsc.VectorSubcoreMesh(core_axis_name="c", subcore_axis_name="s")


@functools.partial(
    pl.kernel,
    mesh=_mesh,
    out_type=jax.ShapeDtypeStruct((_N, _D), jnp.float32),
    scratch_types=[
        pltpu.VMEM((_ROWS_PER_W,), jnp.int32),
        pltpu.VMEM((_CHUNK, _D), jnp.float32),
        pltpu.SemaphoreType.DMA,
    ],
)
def _interleave(x_hbm, perm_hbm, out_hbm, idx_v, rows_v, gsem):
    wid = lax.axis_index("s") * _NC + lax.axis_index("c")
    base = wid * _ROWS_PER_W
    pltpu.sync_copy(perm_hbm.at[pl.ds(base, _ROWS_PER_W)], idx_v)

    def step(c, carry):
        idx_c = idx_v.at[pl.ds(c * _CHUNK, _CHUNK)]
        pltpu.async_copy(x_hbm.at[idx_c], rows_v, gsem).wait()
        pltpu.sync_copy(rows_v, out_hbm.at[pl.ds(base + c * _CHUNK, _CHUNK)])
        return carry

    lax.fori_loop(0, _NCHUNKS, step, 0)


def kernel(x, perm):
    xf = x.reshape(_N, _D)
    out = _interleave(xf, perm)
    return out.reshape(_B, _L, _D)
